# baseline (speedup 1.0000x reference)
import functools

import jax
import jax.numpy as jnp
from jax import lax
from jax.experimental import pallas as pl
from jax.experimental.pallas import tpu as pltpu
from jax.experimental.pallas import tpu_sc as plsc

V = 1000000
D = 64
K = 3
B = 16384
N = B * K
NC, NS = 2, 16
NW = NC * NS
CHUNK = 128
ROWS_PER_W = N // NW
CPW = ROWS_PER_W // CHUNK
PAIR = 2 * D
TV = 16384
S = 507904
NBLK = S // TV


def _tp_body(lo_ref, hi_ref, eye_ref, out_ref):
    out_ref[:, 0:D] = lax.dot_general(
        lo_ref[...], eye_ref[...], (((0,), (0,)), ((), ())),
        preferred_element_type=jnp.float32)
    out_ref[:, D:PAIR] = lax.dot_general(
        hi_ref[...], eye_ref[...], (((0,), (0,)), ((), ())),
        preferred_element_type=jnp.float32)


def _pack(tT):
    eye = jnp.eye(D, dtype=jnp.float32)
    return pl.pallas_call(
        _tp_body,
        grid=(NBLK,),
        in_specs=[
            pl.BlockSpec((D, TV), lambda i: (0, i)),
            pl.BlockSpec((D, TV), lambda i: (0, i + NBLK)),
            pl.BlockSpec((D, D), lambda i: (0, 0)),
        ],
        out_specs=pl.BlockSpec((TV, PAIR), lambda i: (i, 0)),
        out_shape=jax.ShapeDtypeStruct((S, PAIR), jnp.float32),
        compiler_params=pltpu.CompilerParams(
            vmem_limit_bytes=100 * 1024 * 1024),
    )(tT, tT, eye)


def _make_gather(n_rows):
    cpw = n_rows // NW // CHUNK

    def _gather_body(idx_hbm, table_hbm, out_hbm, idx_v, buf0, buf1,
                     gsem0, gsem1, osem0, osem1):
        wid = lax.axis_index("s") * NC + lax.axis_index("c")
        pltpu.sync_copy(idx_hbm.at[wid], idx_v)
        bufs = (buf0, buf1)
        gsems = (gsem0, gsem1)
        osems = (osem0, osem1)
        ocps = [None, None]
        for j in range(cpw):
            b = j & 1
            if ocps[b] is not None:
                ocps[b].wait()
            pltpu.async_copy(table_hbm.at[idx_v.at[j]], bufs[b],
                             gsems[b]).wait()
            ocps[b] = pltpu.async_copy(
                bufs[b],
                out_hbm.at[pl.ds((wid * cpw + j) * CHUNK, CHUNK)],
                osems[b],
            )
        for b in (0, 1):
            if ocps[b] is not None:
                ocps[b].wait()

    return functools.partial(
        pl.kernel,
        mesh=plsc.VectorSubcoreMesh(core_axis_name="c", subcore_axis_name="s"),
        out_type=jax.ShapeDtypeStruct((n_rows, PAIR), jnp.float32),
        scratch_types=[
            pltpu.VMEM((cpw, CHUNK), jnp.int32),
            pltpu.VMEM((CHUNK, PAIR), jnp.float32),
            pltpu.VMEM((CHUNK, PAIR), jnp.float32),
            pltpu.SemaphoreType.DMA,
            pltpu.SemaphoreType.DMA,
            pltpu.SemaphoreType.DMA,
            pltpu.SemaphoreType.DMA,
        ],
        compiler_params=pltpu.CompilerParams(use_tc_tiling_on_sc=True),
    )(_gather_body)


_gather_half = _make_gather(N // 2)


BM = 2048


def _mlp_body(rows_ref, x_ref, w1_ref, b1_ref, w2_ref, b2_ref, out_ref):
    x = x_ref[...]
    total = b1_ref[...]
    for k in range(K):
        pair = rows_ref[:, k * PAIR:(k + 1) * PAIR]
        hi = x[:, k:k + 1] >= S
        e = jnp.where(hi, pair[:, D:PAIR], pair[:, 0:D])
        total = total + jnp.dot(e, w1_ref[k * D:(k + 1) * D, :],
                                preferred_element_type=jnp.float32)
    h = jnp.maximum(total, 0.0)
    o = jnp.sum(h * w2_ref[...], axis=1, keepdims=True) + b2_ref[...]
    out_ref[...] = jax.nn.sigmoid(o)


def _mlp(rows, x, w1, b1, w2row, b2):
    nb = rows.shape[0]
    grid = (nb // BM,)
    return pl.pallas_call(
        _mlp_body,
        grid=grid,
        in_specs=[
            pl.BlockSpec((BM, K * PAIR), lambda i: (i, 0)),
            pl.BlockSpec((BM, K), lambda i: (i, 0)),
            pl.BlockSpec((K * D, D), lambda i: (0, 0)),
            pl.BlockSpec((1, D), lambda i: (0, 0)),
            pl.BlockSpec((1, D), lambda i: (0, 0)),
            pl.BlockSpec((1, 1), lambda i: (0, 0)),
        ],
        out_specs=pl.BlockSpec((BM, 1), lambda i: (i, 0)),
        out_shape=jax.ShapeDtypeStruct((nb, 1), jnp.float32),
    )(rows, x, w1, b1, w2row, b2)


@jax.jit
def kernel(x, table, W1, b1, W2, b2):
    table2 = _pack(table.T)
    flat = x.reshape(-1)
    idx = jnp.where(flat >= S, flat - S, flat)
    b1r, w2r, b2r = b1.reshape(1, D), W2.reshape(1, D), b2.reshape(1, 1)
    outs = []
    half = B // 2
    for h in range(2):
        idx3d = lax.dynamic_slice_in_dim(idx, h * (N // 2), N // 2).reshape(
            NW, (N // 2) // NW // CHUNK, CHUNK)
        wide = _gather_half(idx3d, table2)
        rows = wide.reshape(half, K * PAIR)
        xh = lax.dynamic_slice_in_dim(x, h * half, half)
        outs.append(_mlp(rows, xh, W1, b1r, w2r, b2r))
    return jnp.concatenate(outs, axis=0)


# speedup vs baseline: 1.0065x; 1.0065x over previous
import functools

import jax
import jax.numpy as jnp
from jax import lax
from jax.experimental import pallas as pl
from jax.experimental.pallas import tpu as pltpu
from jax.experimental.pallas import tpu_sc as plsc

V = 1000000
D = 64
K = 3
B = 16384
N = B * K
NC, NS = 2, 16
NW = NC * NS
CHUNK = 128
ROWS_PER_W = N // NW
CPW = ROWS_PER_W // CHUNK
PAIR = 2 * D
TV = 16384
S = 507904
NBLK = S // TV


def _tp_body(lo_ref, hi_ref, eye_ref, out_ref):
    out_ref[:, 0:D] = lax.dot_general(
        lo_ref[...], eye_ref[...], (((0,), (0,)), ((), ())),
        preferred_element_type=jnp.float32)
    out_ref[:, D:PAIR] = lax.dot_general(
        hi_ref[...], eye_ref[...], (((0,), (0,)), ((), ())),
        preferred_element_type=jnp.float32)


def _pack(tT):
    eye = jnp.eye(D, dtype=jnp.float32)
    return pl.pallas_call(
        _tp_body,
        grid=(NBLK,),
        in_specs=[
            pl.BlockSpec((D, TV), lambda i: (0, i)),
            pl.BlockSpec((D, TV), lambda i: (0, i + NBLK)),
            pl.BlockSpec((D, D), lambda i: (0, 0)),
        ],
        out_specs=pl.BlockSpec((TV, PAIR), lambda i: (i, 0)),
        out_shape=jax.ShapeDtypeStruct((S, PAIR), jnp.float32),
        compiler_params=pltpu.CompilerParams(
            vmem_limit_bytes=100 * 1024 * 1024),
    )(tT, tT, eye)


def _make_gather(n_rows):
    cpw = n_rows // NW // CHUNK

    def _gather_body(idx_hbm, table_hbm, out_hbm, idx_v, buf0, buf1,
                     gsem0, gsem1, osem0, osem1):
        wid = lax.axis_index("s") * NC + lax.axis_index("c")
        pltpu.sync_copy(idx_hbm.at[wid], idx_v)
        bufs = (buf0, buf1)
        gsems = (gsem0, gsem1)
        osems = (osem0, osem1)
        ocps = [None, None]
        for j in range(cpw):
            b = j & 1
            if ocps[b] is not None:
                ocps[b].wait()
            pltpu.async_copy(table_hbm.at[idx_v.at[j]], bufs[b],
                             gsems[b]).wait()
            ocps[b] = pltpu.async_copy(
                bufs[b],
                out_hbm.at[pl.ds((wid * cpw + j) * CHUNK, CHUNK)],
                osems[b],
            )
        for b in (0, 1):
            if ocps[b] is not None:
                ocps[b].wait()

    return functools.partial(
        pl.kernel,
        mesh=plsc.VectorSubcoreMesh(core_axis_name="c", subcore_axis_name="s"),
        out_type=jax.ShapeDtypeStruct((n_rows, PAIR), jnp.float32),
        scratch_types=[
            pltpu.VMEM((cpw, CHUNK), jnp.int32),
            pltpu.VMEM((CHUNK, PAIR), jnp.float32),
            pltpu.VMEM((CHUNK, PAIR), jnp.float32),
            pltpu.SemaphoreType.DMA,
            pltpu.SemaphoreType.DMA,
            pltpu.SemaphoreType.DMA,
            pltpu.SemaphoreType.DMA,
        ],
        compiler_params=pltpu.CompilerParams(use_tc_tiling_on_sc=True),
    )(_gather_body)


_gather_all = _make_gather(N)


BM = 2048


def _mlp_body(rows_ref, x_ref, w1_ref, b1_ref, w2_ref, b2_ref, out_ref):
    x = x_ref[...]
    total = b1_ref[...]
    for k in range(K):
        pair = rows_ref[:, k * PAIR:(k + 1) * PAIR]
        hi = x[:, k:k + 1] >= S
        e = jnp.where(hi, pair[:, D:PAIR], pair[:, 0:D])
        total = total + jnp.dot(e, w1_ref[k * D:(k + 1) * D, :],
                                preferred_element_type=jnp.float32)
    h = jnp.maximum(total, 0.0)
    o = jnp.sum(h * w2_ref[...], axis=1, keepdims=True) + b2_ref[...]
    out_ref[...] = jax.nn.sigmoid(o)


def _mlp(rows, x, w1, b1, w2row, b2):
    nb = rows.shape[0]
    grid = (nb // BM,)
    return pl.pallas_call(
        _mlp_body,
        grid=grid,
        in_specs=[
            pl.BlockSpec((BM, K * PAIR), lambda i: (i, 0)),
            pl.BlockSpec((BM, K), lambda i: (i, 0)),
            pl.BlockSpec((K * D, D), lambda i: (0, 0)),
            pl.BlockSpec((1, D), lambda i: (0, 0)),
            pl.BlockSpec((1, D), lambda i: (0, 0)),
            pl.BlockSpec((1, 1), lambda i: (0, 0)),
        ],
        out_specs=pl.BlockSpec((BM, 1), lambda i: (i, 0)),
        out_shape=jax.ShapeDtypeStruct((nb, 1), jnp.float32),
    )(rows, x, w1, b1, w2row, b2)


@jax.jit
def kernel(x, table, W1, b1, W2, b2):
    table2 = _pack(table.T)
    flat = x.reshape(-1)
    idx3d = jnp.where(flat >= S, flat - S, flat).reshape(NW, CPW, CHUNK)
    wide = _gather_all(idx3d, table2)
    rows = wide.reshape(B, K * PAIR)
    return _mlp(rows, x, W1, b1.reshape(1, D), W2.reshape(1, D),
                b2.reshape(1, 1))
